# Initial kernel scaffold; baseline (speedup 1.0000x reference)
import jax
import jax.numpy as jnp
from jax.experimental import pallas as pl


def kernel(voted_logit, target, t_hidden_states, t_att_matrices, s_hidden_states, s_att_matrices, teacher_cs_token_align, student_cs_token_align, cs_token_align_len):
    raise NotImplementedError("write your pallas kernel here")


# speedup vs baseline: 1.1971x; 1.1971x over previous
import functools

import jax
import jax.numpy as jnp
from jax import lax
from jax.experimental import pallas as pl
from jax.experimental.pallas import tpu as pltpu
from jax.experimental.pallas import tpu_sc as plsc

ALPHA_C = 0.1
BETA_C = 0.1

L, B, H, S, D = 4, 4, 12, 512, 768
MAXCS = 8
NW = 32
ATT_ITEMS = B * MAXCS * L * H
HID_ITEMS = B * MAXCS * L
ATT_SLOTS = ATT_ITEMS // NW
HID_SLOTS = HID_ITEMS // NW
ATT_META_W = ATT_SLOTS * 8 + 16
HID_META_W = HID_SLOTS * 8 + 16
N_ATT_CHUNKS = S // 16
N_HID_CHUNKS = D // 16


def _sc_partials(t_att_flat, s_att_flat, t_hid_flat, s_hid_flat,
                 meta_att, meta_hid):
    mesh = plsc.VectorSubcoreMesh(core_axis_name="c", subcore_axis_name="s")

    @functools.partial(
        pl.kernel,
        mesh=mesh,
        out_type=jax.ShapeDtypeStruct((NW, 32), jnp.float32),
        compiler_params=pltpu.CompilerParams(needs_layout_passes=False),
        scratch_types=[
            pltpu.VMEM((ATT_META_W,), jnp.int32),
            pltpu.VMEM((HID_META_W,), jnp.int32),
            pltpu.VMEM((16, S), jnp.float32),
            pltpu.VMEM((16, S), jnp.float32),
            pltpu.VMEM((16, D), jnp.float32),
            pltpu.VMEM((16, D), jnp.float32),
            pltpu.VMEM((S,), jnp.float32),
            pltpu.VMEM((S,), jnp.float32),
            pltpu.VMEM((D,), jnp.float32),
            pltpu.VMEM((D,), jnp.float32),
            pltpu.VMEM((16,), jnp.float32),
            pltpu.VMEM((16,), jnp.float32),
            pltpu.VMEM((32,), jnp.float32),
            pltpu.SemaphoreType.DMA,
            pltpu.SemaphoreType.DMA,
        ],
    )
    def k(t_att_hbm, s_att_hbm, t_hid_hbm, s_hid_hbm, meta_att_hbm,
          meta_hid_hbm, out_hbm, meta_att_v, meta_hid_v, buf_t, buf_s,
          buf_ht, buf_hs, pool_t, pool_s, pool_ht, pool_hs, attn_acc,
          hidn_acc, out_v, sem0, sem1):
        wid = lax.axis_index("s") * 2 + lax.axis_index("c")
        iota16 = lax.iota(jnp.int32, 16)
        zero16 = jnp.zeros((16,), jnp.float32)

        pltpu.sync_copy(meta_att_hbm.at[wid], meta_att_v)
        pltpu.sync_copy(meta_hid_hbm.at[wid], meta_hid_v)
        attn_acc[...] = zero16
        hidn_acc[...] = zero16

        def lane(vec, f):
            return jnp.sum(jnp.where(iota16 == f, vec, 0))

        def lane_f(vec, f):
            vf = plsc.bitcast(vec, jnp.float32)
            return jnp.sum(jnp.where(iota16 == f, vf, 0.0))

        def att_item(j, carry):
            mvec = meta_att_v[pl.ds(j * 8, 16)]
            t_row = lane(mvec, 0)
            t_cnt = lane(mvec, 1)
            s_row = lane(mvec, 2)
            s_cnt = lane(mvec, 3)
            valid = lane(mvec, 4)

            @pl.when(valid > 0)
            def _():
                cp_t = pltpu.async_copy(t_att_hbm.at[t_row + iota16], buf_t,
                                        sem0)
                cp_s = pltpu.async_copy(s_att_hbm.at[s_row + iota16], buf_s,
                                        sem1)
                for c in range(N_ATT_CHUNKS):
                    pool_t[pl.ds(16 * c, 16)] = zero16
                    pool_s[pl.ds(16 * c, 16)] = zero16
                cp_t.wait()

                def row_t(i, c2):
                    for c in range(N_ATT_CHUNKS):
                        x = buf_t[i, pl.ds(16 * c, 16)]
                        x = jnp.where(x <= -100.0, 0.0, x)
                        plsc.addupdate(pool_t.at[pl.ds(16 * c, 16)], x)
                    return c2
                lax.fori_loop(0, t_cnt, row_t, 0)
                cp_s.wait()

                def row_s(i, c2):
                    for c in range(N_ATT_CHUNKS):
                        x = buf_s[i, pl.ds(16 * c, 16)]
                        x = jnp.where(x <= -100.0, 0.0, x)
                        plsc.addupdate(pool_s.at[pl.ds(16 * c, 16)], x)
                    return c2
                lax.fori_loop(0, s_cnt, row_s, 0)

                inv_t = lane_f(mvec, 5)
                inv_s = lane_f(mvec, 6)
                acc = zero16
                for c in range(N_ATT_CHUNKS):
                    dlt = (pool_t[pl.ds(16 * c, 16)] * inv_t
                           - pool_s[pl.ds(16 * c, 16)] * inv_s)
                    acc = acc + dlt * dlt
                attn_acc[...] = attn_acc[...] + acc
            return carry

        lax.fori_loop(0, ATT_SLOTS, att_item, 0)

        def hid_item(j, carry):
            mvec = meta_hid_v[pl.ds(j * 8, 16)]
            t_row = lane(mvec, 0)
            t_cnt = lane(mvec, 1)
            s_row = lane(mvec, 2)
            s_cnt = lane(mvec, 3)
            valid = lane(mvec, 4)

            @pl.when(valid > 0)
            def _():
                cp_t = pltpu.async_copy(t_hid_hbm.at[t_row + iota16], buf_ht,
                                        sem0)
                cp_s = pltpu.async_copy(s_hid_hbm.at[s_row + iota16], buf_hs,
                                        sem1)
                for c in range(N_HID_CHUNKS):
                    pool_ht[pl.ds(16 * c, 16)] = zero16
                    pool_hs[pl.ds(16 * c, 16)] = zero16
                cp_t.wait()

                def row_t(i, c2):
                    for c in range(N_HID_CHUNKS):
                        plsc.addupdate(pool_ht.at[pl.ds(16 * c, 16)],
                                       buf_ht[i, pl.ds(16 * c, 16)])
                    return c2
                lax.fori_loop(0, t_cnt, row_t, 0)
                cp_s.wait()

                def row_s(i, c2):
                    for c in range(N_HID_CHUNKS):
                        plsc.addupdate(pool_hs.at[pl.ds(16 * c, 16)],
                                       buf_hs[i, pl.ds(16 * c, 16)])
                    return c2
                lax.fori_loop(0, s_cnt, row_s, 0)

                inv_t = lane_f(mvec, 5)
                inv_s = lane_f(mvec, 6)
                acc = zero16
                for c in range(N_HID_CHUNKS):
                    dlt = (pool_ht[pl.ds(16 * c, 16)] * inv_t
                           - pool_hs[pl.ds(16 * c, 16)] * inv_s)
                    acc = acc + dlt * dlt
                hidn_acc[...] = hidn_acc[...] + acc
            return carry

        lax.fori_loop(0, HID_SLOTS, hid_item, 0)

        out_v[pl.ds(0, 16)] = attn_acc[...]
        out_v[pl.ds(16, 16)] = hidn_acc[...]
        pltpu.sync_copy(out_v, out_hbm.at[wid])

    return k(t_att_flat, s_att_flat, t_hid_flat, s_hid_flat, meta_att,
             meta_hid)


def _combine_kernel(partials_ref, logit_ref, onehot_ref, lenf_ref,
                    hidn_ref, attn_ref, pred_ref):
    p = partials_ref[...]
    attn_sum = jnp.sum(p[:, :16])
    hidn_sum = jnp.sum(p[:, 16:])
    nv = jnp.sum(lenf_ref[...])
    hidn_ref[...] = jnp.reshape(ALPHA_C * hidn_sum / (nv * L * D), (1, 1))
    attn_ref[...] = jnp.reshape(BETA_C * attn_sum / (nv * L * H * S), (1, 1))
    logit = logit_ref[...]
    m = jnp.max(logit, axis=-1, keepdims=True)
    lse = jnp.log(jnp.sum(jnp.exp(logit - m), axis=-1, keepdims=True)) + m
    logp = logit - lse
    pred_ref[...] = jnp.reshape(-jnp.sum(logp * onehot_ref[...]) / B, (1, 1))


def kernel(voted_logit, target, t_hidden_states, t_att_matrices,
           s_hidden_states, s_att_matrices, teacher_cs_token_align,
           student_cs_token_align, cs_token_align_len):
    nc = voted_logit.shape[-1]

    t_att_flat = t_att_matrices.reshape(L * B * H * S, S)
    s_att_flat = s_att_matrices.reshape(L * B * H * S, S)
    t_hid_flat = t_hidden_states.reshape(L * B * S, D)
    s_hid_flat = s_hidden_states.reshape(L * B * S, D)

    ts = teacher_cs_token_align[:, :, 0]
    te = teacher_cs_token_align[:, :, 1]
    ss = student_cs_token_align[:, :, 0]
    se = student_cs_token_align[:, :, 1]
    valid = (jnp.arange(MAXCS)[None, :]
             < cs_token_align_len[:, None]).astype(jnp.int32)

    b_idx = jnp.arange(B)[:, None, None, None]
    c_idx = jnp.arange(MAXCS)[None, :, None, None]
    l_idx = jnp.arange(L)[None, None, :, None]
    h_idx = jnp.arange(H)[None, None, None, :]
    att_t_row = ((l_idx * B + b_idx) * H + h_idx) * S + ts[:, :, None, None]
    att_s_row = ((l_idx * B + b_idx) * H + h_idx) * S + ss[:, :, None, None]
    bc_bcast = jnp.broadcast_to((te - ts)[:, :, None, None], att_t_row.shape)
    sc_bcast = jnp.broadcast_to((se - ss)[:, :, None, None], att_t_row.shape)
    v_bcast = jnp.broadcast_to(valid[:, :, None, None], att_t_row.shape)
    inv_tc = lax.bitcast_convert_type(
        1.0 / (te - ts).astype(jnp.float32), jnp.int32)
    inv_sc = lax.bitcast_convert_type(
        1.0 / (se - ss).astype(jnp.float32), jnp.int32)
    itc_b = jnp.broadcast_to(inv_tc[:, :, None, None], att_t_row.shape)
    isc_b = jnp.broadcast_to(inv_sc[:, :, None, None], att_t_row.shape)
    att_fields = jnp.stack(
        [att_t_row, bc_bcast, att_s_row, sc_bcast, v_bcast,
         itc_b, isc_b,
         jnp.zeros_like(att_t_row)], axis=-1).reshape(ATT_ITEMS, 8)
    att_fields = att_fields.reshape(ATT_SLOTS, NW, 8).transpose(1, 0, 2)
    meta_att = jnp.zeros((NW, ATT_META_W), jnp.int32)
    meta_att = meta_att.at[:, :ATT_SLOTS * 8].set(
        att_fields.reshape(NW, ATT_SLOTS * 8))

    b3 = jnp.arange(B)[:, None, None]
    c3 = jnp.arange(MAXCS)[None, :, None]
    l3 = jnp.arange(L)[None, None, :]
    hid_t_row = (l3 * B + b3) * S + ts[:, :, None]
    hid_s_row = (l3 * B + b3) * S + ss[:, :, None]
    bc3 = jnp.broadcast_to((te - ts)[:, :, None], hid_t_row.shape)
    sc3 = jnp.broadcast_to((se - ss)[:, :, None], hid_t_row.shape)
    v3 = jnp.broadcast_to(valid[:, :, None], hid_t_row.shape)
    itc3 = jnp.broadcast_to(inv_tc[:, :, None], hid_t_row.shape)
    isc3 = jnp.broadcast_to(inv_sc[:, :, None], hid_t_row.shape)
    hid_fields = jnp.stack(
        [hid_t_row, bc3, hid_s_row, sc3, v3,
         itc3, isc3,
         jnp.zeros_like(hid_t_row)], axis=-1).reshape(HID_ITEMS, 8)
    hid_fields = hid_fields.reshape(HID_SLOTS, NW, 8).transpose(1, 0, 2)
    meta_hid = jnp.zeros((NW, HID_META_W), jnp.int32)
    meta_hid = meta_hid.at[:, :HID_SLOTS * 8].set(
        hid_fields.reshape(NW, HID_SLOTS * 8))

    partials = _sc_partials(t_att_flat, s_att_flat, t_hid_flat, s_hid_flat,
                            meta_att, meta_hid)

    onehot = jax.nn.one_hot(target, nc, dtype=jnp.float32)
    lenf = cs_token_align_len.astype(jnp.float32).reshape(1, B)
    hidn, attn, pred = pl.pallas_call(
        _combine_kernel,
        out_shape=[jax.ShapeDtypeStruct((1, 1), jnp.float32)] * 3,
    )(partials, voted_logit, onehot, lenf)
    return (hidn[0, 0], attn[0, 0], pred[0, 0])
